# baseline (speedup 1.0000x reference)
import functools

import jax
import jax.numpy as jnp
from jax import lax
from jax.experimental import pallas as pl
from jax.experimental.pallas import tpu as pltpu, tpu_sc as plsc

N = 4096
BH = BW = 32
GRID = N // BH
NNZ = 1638
NW = 32
PER_W = 52
RB = 13
NROUNDS = PER_W // RB
BLK = BH * BW

_mesh = plsc.VectorSubcoreMesh(core_axis_name="c", subcore_axis_name="s")


@functools.partial(
    pl.kernel,
    out_type=jax.ShapeDtypeStruct((NNZ * BLK // 128, 128), jnp.float32),
    mesh=_mesh,
    scratch_types=[
        pltpu.VMEM((128,), jnp.int32),
        pltpu.VMEM((128,), jnp.int32),
        pltpu.VMEM((2 * RB, BH, 4 * BW), jnp.float32),
        pltpu.VMEM((RB * BLK // 128, 128), jnp.float32),
        pltpu.SemaphoreType.DMA,
        pltpu.SemaphoreType.DMA,
        pltpu.SemaphoreType.DMA,
    ],
)
def _gather_blocks(x, r2d, c2d, out, rows_v, cols_v, wide, stage,
                   sem_in0, sem_in1, sem_out):
    wid = lax.axis_index("s") * 2 + lax.axis_index("c")
    pltpu.sync_copy(r2d.at[wid], rows_v)
    pltpu.sync_copy(c2d.at[wid], cols_v)

    rc = []
    for j in range((PER_W + 15) // 16):
        r16 = rows_v[pl.ds(j * 16, 16)]
        c16 = cols_v[pl.ds(j * 16, 16)]
        for k in range(16):
            if j * 16 + k >= PER_W:
                break
            rc.append((r16[k], c16[k]))

    def in_copy(g, m):
        r, c = rc[g * RB + m]
        return pltpu.make_async_copy(
            x.at[pl.ds(r * BH, BH), pl.ds((c >> 2) * (4 * BW), 4 * BW)],
            wide.at[(g % 2) * RB + m], sem_in1 if g % 2 else sem_in0)

    def out_copy(g):
        row0 = (wid * PER_W + g * RB) * (BLK // 128)
        return pltpu.make_async_copy(
            stage, out.at[pl.ds(row0, RB * BLK // 128)], sem_out)

    for m in range(RB):
        in_copy(0, m).start()
    for g in range(NROUNDS):
        if g + 1 < NROUNDS:
            for m in range(RB):
                in_copy(g + 1, m).start()
        for m in range(RB):
            in_copy(g, m).wait()

        offs = [(c & 3) * BW for _, c in rc[g * RB:(g + 1) * RB]]
        slot0 = (g % 2) * RB

        def extract_row(i, _):
            srow = i >> 2
            scol = (i & 3) * BW
            for m in range(RB):
                src = wide.at[slot0 + m]
                for h in (0, 16):
                    stage[m * 8 + srow, pl.ds(scol + h, 16)] = (
                        src[i, pl.ds(offs[m] + h, 16)])
            return _

        valid = (wid < NW - 1) if g >= 2 else None
        if valid is None:
            lax.fori_loop(0, BH, extract_row, 0, unroll=4)
            out_copy(g).start()
            out_copy(g).wait()
        else:
            @pl.when(valid)
            def _():
                lax.fori_loop(0, BH, extract_row, 0, unroll=4)
                out_copy(g).start()
                out_copy(g).wait()


def kernel(x, active_indices):
    ai = active_indices.astype(jnp.int32)
    pad = jnp.zeros((NW * PER_W, 2), jnp.int32).at[:NNZ].set(ai)
    r2d = jnp.zeros((NW, 128), jnp.int32).at[:, :PER_W].set(
        pad[:, 0].reshape(NW, PER_W))
    c2d = jnp.zeros((NW, 128), jnp.int32).at[:, :PER_W].set(
        pad[:, 1].reshape(NW, PER_W))
    out2d = _gather_blocks(x, r2d, c2d)
    return out2d.reshape(NNZ, BH, BW)
